# baseline (speedup 1.0000x reference)
import functools

import jax
import jax.numpy as jnp
from jax import lax
from jax.experimental import pallas as pl
from jax.experimental.pallas import tpu as pltpu
from jax.experimental.pallas import tpu_sc as plsc

_NC = 2
_NS = 16
_NW = _NC * _NS

_ROW = 128
_IDX_PER_STREAM = 128
_HIST_PAD = 56


def _transpose_tc(w, vocab_pad, block_w):
    units = w.shape[0]

    def body(in_ref, out_ref):
        x = in_ref[...].T
        out_ref[...] = jnp.concatenate(
            [x, jnp.zeros((x.shape[0], _ROW - units), x.dtype)], axis=1)

    return pl.pallas_call(
        body,
        grid=(vocab_pad // block_w,),
        in_specs=[pl.BlockSpec((units, block_w), lambda i: (0, i))],
        out_specs=pl.BlockSpec((block_w, _ROW), lambda i: (i, 0)),
        out_shape=jax.ShapeDtypeStruct((vocab_pad, _ROW), w.dtype),
    )(w)


def _make_gather(vocab_pad, b_pad):
    b_per_w = b_pad // _NW
    sup = 7168
    sup_rows = sup // _IDX_PER_STREAM
    n_sup = b_per_w // sup
    chunk = 896
    n_chunks = sup // chunk
    spc = chunk // _IDX_PER_STREAM

    mesh = plsc.VectorSubcoreMesh(core_axis_name="c", subcore_axis_name="s")

    @functools.partial(
        pl.kernel,
        mesh=mesh,
        compiler_params=pltpu.CompilerParams(use_tc_tiling_on_sc=True),
        out_type=jax.ShapeDtypeStruct((b_pad, _ROW), jnp.float32),
        scratch_types=[
            pltpu.VMEM((sup_rows, _IDX_PER_STREAM), jnp.int32),
            pltpu.VMEM((chunk, _ROW), jnp.float32),
            pltpu.SemaphoreType.DMA,
        ],
    )
    def gather_kernel(table_hbm, idx_hbm, out_hbm, idx_v, rows_v, gat_sem):
        wid = lax.axis_index("s") * _NC + lax.axis_index("c")
        base_row = wid * (b_per_w // _IDX_PER_STREAM)

        def sup_body(s, _):
            pltpu.sync_copy(
                idx_hbm.at[pl.ds(base_row + s * sup_rows, sup_rows)], idx_v)
            for c in range(n_chunks):
                for j in range(spc):
                    pltpu.async_copy(
                        table_hbm.at[idx_v.at[c * spc + j]],
                        rows_v.at[pl.ds(j * _IDX_PER_STREAM,
                                        _IDX_PER_STREAM)],
                        gat_sem)
                for j in range(spc):
                    pltpu.make_async_copy(
                        table_hbm.at[idx_v.at[c * spc + j]],
                        rows_v.at[pl.ds(j * _IDX_PER_STREAM,
                                        _IDX_PER_STREAM)],
                        gat_sem).wait()
                pltpu.sync_copy(
                    rows_v,
                    out_hbm.at[pl.ds(
                        wid * b_per_w + s * sup + c * chunk, chunk)])
            return ()

        lax.fori_loop(0, n_sup, sup_body, (), unroll=False)

    return gather_kernel


def kernel(inputs, kernel):
    units, vocab = kernel.shape
    batch, hist = inputs.shape

    vocab_pad = 102400
    w = jnp.pad(kernel, ((0, 0), (0, vocab_pad - vocab)))
    table = _transpose_tc(w, vocab_pad, block_w=4096)

    idx2 = jnp.pad(inputs.astype(jnp.int32), ((0, 0), (0, _HIST_PAD - hist)))
    b_pad = batch * _HIST_PAD
    idx2d = idx2.reshape(b_pad // _IDX_PER_STREAM, _IDX_PER_STREAM)

    out = _make_gather(vocab_pad, b_pad)(table, idx2d)
    return out.reshape(batch, _HIST_PAD, _ROW)[:, :hist, :units]


# speedup vs baseline: 5.3566x; 5.3566x over previous
import functools

import jax
import jax.numpy as jnp
from jax import lax
from jax.experimental import pallas as pl
from jax.experimental.pallas import tpu as pltpu
from jax.experimental.pallas import tpu_sc as plsc

_NC = 2
_NS = 16
_NW = _NC * _NS

_IDX_PER_STREAM = 128
_STREAMS_PER_CHUNK = 8
_CHUNK = _IDX_PER_STREAM * _STREAMS_PER_CHUNK


def _make_transpose(units, vocab_pad):
    v_per_w = vocab_pad // _NW
    cchunk = 800
    n_chunks = v_per_w // cchunk

    mesh = plsc.VectorSubcoreMesh(core_axis_name="c", subcore_axis_name="s")

    @functools.partial(
        pl.kernel,
        mesh=mesh,
        compiler_params=pltpu.CompilerParams(use_tc_tiling_on_sc=False,
                                             needs_layout_passes=False),
        out_type=jax.ShapeDtypeStruct((vocab_pad, units), jnp.float32),
        scratch_types=[
            pltpu.VMEM((units * cchunk,), jnp.float32),
            pltpu.VMEM((cchunk, units), jnp.float32),
            pltpu.SemaphoreType.DMA,
        ],
    )
    def transpose_kernel(w_hbm, table_hbm, in_v, out_v, sem):
        wid = lax.axis_index("s") * _NC + lax.axis_index("c")
        col_base = wid * v_per_w
        iota16 = lax.iota(jnp.int32, 16)
        stride_idx = iota16 * cchunk

        for ch in range(n_chunks):
            col0 = col_base + ch * cchunk
            for u in range(units):
                pltpu.async_copy(
                    w_hbm.at[pl.ds(u * vocab_pad + col0, cchunk)],
                    in_v.at[pl.ds(u * cchunk, cchunk)],
                    sem)
            for u in range(units):
                pltpu.make_async_copy(
                    w_hbm.at[pl.ds(u * vocab_pad + col0, cchunk)],
                    in_v.at[pl.ds(u * cchunk, cchunk)],
                    sem).wait()

            def body(i, _):
                c = i * 4
                for dc in range(4):
                    for g in range(units // 16):
                        src = stride_idx + (g * 16 * cchunk + c + dc)
                        vals = plsc.load_gather(in_v, [src])
                        rows = jnp.full((16,), c + dc, jnp.int32)
                        cols = iota16 + g * 16
                        plsc.store_scatter(out_v, [rows, cols], vals)
                return ()

            lax.fori_loop(0, cchunk // 4, body, (), unroll=False)

            pltpu.sync_copy(out_v, table_hbm.at[pl.ds(col0, cchunk)])

    return transpose_kernel


def _make_gather(vocab_pad, units, b):
    b_per_w = b // _NW
    n_chunks = b_per_w // _CHUNK
    chunk_rows = _CHUNK // _IDX_PER_STREAM

    mesh = plsc.VectorSubcoreMesh(core_axis_name="c", subcore_axis_name="s")

    @functools.partial(
        pl.kernel,
        mesh=mesh,
        compiler_params=pltpu.CompilerParams(use_tc_tiling_on_sc=False),
        out_type=jax.ShapeDtypeStruct((b, units), jnp.float32),
        scratch_types=[
            pltpu.VMEM((chunk_rows, _IDX_PER_STREAM), jnp.int32),
            pltpu.VMEM((_CHUNK, units), jnp.float32),
            pltpu.SemaphoreType.DMA,
        ],
    )
    def gather_kernel(table_hbm, idx_hbm, out_hbm, idx_v, rows_v, gat_sem):
        wid = lax.axis_index("s") * _NC + lax.axis_index("c")
        base_row = wid * (b_per_w // _IDX_PER_STREAM)

        def body(g, _):
            pltpu.sync_copy(
                idx_hbm.at[pl.ds(base_row + g * chunk_rows, chunk_rows)],
                idx_v)
            for j in range(_STREAMS_PER_CHUNK):
                pltpu.async_copy(
                    table_hbm.at[idx_v.at[j]],
                    rows_v.at[pl.ds(j * _IDX_PER_STREAM, _IDX_PER_STREAM)],
                    gat_sem)
            for j in range(_STREAMS_PER_CHUNK):
                pltpu.make_async_copy(
                    table_hbm.at[idx_v.at[j]],
                    rows_v.at[pl.ds(j * _IDX_PER_STREAM, _IDX_PER_STREAM)],
                    gat_sem).wait()
            pltpu.sync_copy(
                rows_v,
                out_hbm.at[pl.ds(wid * b_per_w + g * _CHUNK, _CHUNK)])
            return ()

        lax.fori_loop(0, n_chunks, body, (), unroll=False)

    return gather_kernel


def kernel(inputs, kernel):
    units, vocab = kernel.shape
    batch, hist = inputs.shape
    b = batch * hist

    vocab_pad = 102400
    w1d = jnp.pad(kernel, ((0, 0), (0, vocab_pad - vocab))).reshape(-1)
    table = _make_transpose(units, vocab_pad)(w1d)

    idx = inputs.astype(jnp.int32).reshape(b // _IDX_PER_STREAM,
                                           _IDX_PER_STREAM)
    out = _make_gather(vocab_pad, units, b)(table, idx)
    return out.reshape(batch, hist, units)


# speedup vs baseline: 6.3814x; 1.1913x over previous
import functools

import jax
import jax.numpy as jnp
from jax import lax
from jax.experimental import pallas as pl
from jax.experimental.pallas import tpu as pltpu
from jax.experimental.pallas import tpu_sc as plsc

_NC = 2
_NS = 16
_NW = _NC * _NS

_IDX_PER_STREAM = 128
_STREAMS_PER_CHUNK = 5
_CHUNK = _IDX_PER_STREAM * _STREAMS_PER_CHUNK


def _transpose_tc(w, vocab_pad, block_w):
    units = w.shape[0]

    def body(in_ref, out_ref):
        out_ref[...] = in_ref[...].T

    return pl.pallas_call(
        body,
        grid=(vocab_pad // block_w,),
        in_specs=[pl.BlockSpec((units, block_w), lambda i: (0, i))],
        out_specs=pl.BlockSpec((block_w, units), lambda i: (i, 0)),
        out_shape=jax.ShapeDtypeStruct((vocab_pad, units), w.dtype),
    )(w)


def _make_gather(vocab_pad, units, b):
    b_per_w = b // _NW
    n_chunks = b_per_w // _CHUNK
    n_pairs = n_chunks // 2
    rows = _STREAMS_PER_CHUNK

    mesh = plsc.VectorSubcoreMesh(core_axis_name="c", subcore_axis_name="s")

    @functools.partial(
        pl.kernel,
        mesh=mesh,
        compiler_params=pltpu.CompilerParams(use_tc_tiling_on_sc=False),
        out_type=jax.ShapeDtypeStruct((b, units), jnp.float32),
        scratch_types=[
            pltpu.VMEM((2, rows, _IDX_PER_STREAM), jnp.int32),
            pltpu.VMEM((2, _CHUNK, units), jnp.float32),
            pltpu.SemaphoreType.DMA((2,)),
        ],
    )
    def gather_kernel(table_hbm, idx_hbm, out_hbm, idx_v, rows_v, gat_sem):
        wid = lax.axis_index("s") * _NC + lax.axis_index("c")
        base_row = wid * (b_per_w // _IDX_PER_STREAM)
        out_base = wid * b_per_w

        def load_idx(g, h):
            pltpu.sync_copy(
                idx_hbm.at[pl.ds(base_row + g * rows, rows)], idx_v.at[h])

        def fire(h):
            for j in range(_STREAMS_PER_CHUNK):
                pltpu.async_copy(
                    table_hbm.at[idx_v.at[h, j]],
                    rows_v.at[h, pl.ds(j * _IDX_PER_STREAM, _IDX_PER_STREAM)],
                    gat_sem.at[h])

        def drain(h):
            for j in range(_STREAMS_PER_CHUNK):
                pltpu.make_async_copy(
                    table_hbm.at[idx_v.at[h, j]],
                    rows_v.at[h, pl.ds(j * _IDX_PER_STREAM, _IDX_PER_STREAM)],
                    gat_sem.at[h]).wait()

        def write(g, h):
            pltpu.sync_copy(rows_v.at[h],
                            out_hbm.at[pl.ds(out_base + g * _CHUNK, _CHUNK)])

        load_idx(0, 0)
        fire(0)

        def pair_body(k, _):
            g = 2 * k
            load_idx(g + 1, 1)
            drain(0)
            fire(1)
            write(g, 0)

            @pl.when(k + 1 < n_pairs)
            def _():
                load_idx(g + 2, 0)
            drain(1)

            @pl.when(k + 1 < n_pairs)
            def _():
                fire(0)
            write(g + 1, 1)
            return ()

        lax.fori_loop(0, n_pairs, pair_body, (), unroll=False)

    return gather_kernel


def kernel(inputs, kernel):
    units, vocab = kernel.shape
    batch, hist = inputs.shape
    b = batch * hist

    vocab_pad = 102400
    w = jnp.pad(kernel, ((0, 0), (0, vocab_pad - vocab)))
    table = _transpose_tc(w, vocab_pad, block_w=4096)

    idx = inputs.astype(jnp.int32).reshape(b // _IDX_PER_STREAM,
                                           _IDX_PER_STREAM)
    out = _make_gather(vocab_pad, units, b)(table, idx)
    return out.reshape(batch, hist, units)
